# baseline (speedup 1.0000x reference)
import jax
import jax.numpy as jnp
from jax import lax
from jax.experimental import pallas as pl
from jax.experimental.pallas import tpu as pltpu
from jax.experimental.pallas import tpu_sc as plsc

N = 10000
D = 128
E = 320000
NP = 10240
NC = 2
NS = 16
NW = NC * NS
CH = 128
NCH = 80
HP = 40
EPAD = NW * NCH * CH
RPS = NP // NS

_MESH = plsc.VectorSubcoreMesh(core_axis_name="c", subcore_axis_name="s",
                               num_cores=NC, num_subcores=NS)


def _fill(ref, n, value):
    def body(i, _):
        ref[pl.ds(i * 16, 16)] = jnp.full((16,), value, jnp.float32)
        return 0
    lax.fori_loop(0, n // 16, body, 0)


def _deg_body(dst_hbm, out_hbm, acc, idxv, ones, buf, sd):
    c = lax.axis_index("c")
    s = lax.axis_index("s")
    wid = c * NS + s
    _fill(ones, CH, 1.0)
    _fill(buf, RPS, 0.0)
    pltpu.sync_copy(buf, acc.at[pl.ds(s * RPS, RPS)])
    plsc.subcore_barrier()
    pltpu.sync_copy(dst_hbm.at[wid], idxv)

    def body(j, _):
        pltpu.async_copy(ones, acc.at[idxv.at[j]], sd, add=True)
        return 0
    lax.fori_loop(0, NCH, body, 0)

    def drain(j, _):
        pltpu.make_async_copy(ones, acc.at[idxv.at[j]], sd).wait()
        return 0
    lax.fori_loop(0, NCH, drain, 0)
    plsc.subcore_barrier()
    pltpu.sync_copy(acc.at[pl.ds(s * RPS, RPS)], buf)
    pltpu.sync_copy(buf, out_hbm.at[c, pl.ds(s * RPS, RPS)])


_deg_call = pl.kernel(
    _deg_body,
    out_type=jax.ShapeDtypeStruct((NC, NP), jnp.float32),
    mesh=_MESH,
    scratch_types=[
        pltpu.VMEM_SHARED((NP,), jnp.float32),
        pltpu.VMEM((NCH, CH), jnp.int32),
        pltpu.VMEM((CH,), jnp.float32),
        pltpu.VMEM((RPS,), jnp.float32),
        pltpu.SemaphoreType.DMA,
    ],
)


def _gs_body(y_hbm, src_hbm, dst_hbm, out_hbm, acc, sidx, didx, ra, rb,
             sa, sb, ta, tb):
    c = lax.axis_index("c")
    s = lax.axis_index("s")
    wid = c * NS + s

    def zrow(i, _):
        ra[i // 8, pl.ds((i % 8) * 16, 16)] = jnp.zeros((16,), jnp.float32)
        return 0
    lax.fori_loop(0, CH * 8, zrow, 0)
    for t in range(RPS // CH):
        pltpu.async_copy(ra, acc.at[pl.ds(s * RPS + t * CH, CH)], sa)
    for t in range(RPS // CH):
        pltpu.make_async_copy(ra, acc.at[pl.ds(s * RPS + t * CH, CH)],
                              sa).wait()
    plsc.subcore_barrier()

    def gather(j, buf, sem):
        pltpu.async_copy(y_hbm.at[sidx.at[j, pl.ds(0, 64)]],
                         buf.at[pl.ds(0, 64)], sem)
        pltpu.async_copy(y_hbm.at[sidx.at[j, pl.ds(64, 64)]],
                         buf.at[pl.ds(64, 64)], sem)

    def gwait(j, buf, sem):
        pltpu.make_async_copy(y_hbm.at[sidx.at[j]], buf, sem).wait()

    for p in range(NCH // HP):
        pltpu.sync_copy(src_hbm.at[wid, pl.ds(p * HP, HP)], sidx)
        pltpu.sync_copy(dst_hbm.at[wid, pl.ds(p * HP, HP)], didx)
        gather(0, ra, sa)

        def body(i, _):
            j0 = 2 * i
            j1 = j0 + 1
            gwait(j0, ra, sa)
            gather(j1, rb, sb)
            pltpu.sync_copy(ra, acc.at[didx.at[j0]], add=True)

            @pl.when(j1 + 1 < HP)
            def _():
                gather(j1 + 1, ra, sa)
            gwait(j1, rb, sb)
            pltpu.sync_copy(rb, acc.at[didx.at[j1]], add=True)
            return 0
        lax.fori_loop(0, HP // 2, body, 0)
    plsc.subcore_barrier()
    bufs = [ra, rb]
    rsems = [sa, sb]
    wsems = [ta, tb]
    nt = RPS // CH
    pltpu.async_copy(acc.at[pl.ds(s * RPS, CH)], ra, sa)
    for t in range(nt):
        b = bufs[t % 2]
        nb = bufs[(t + 1) % 2]
        row = s * RPS + t * CH
        pltpu.make_async_copy(acc.at[pl.ds(row, CH)], b, rsems[t % 2]).wait()
        if t + 1 < nt:
            if t >= 1:
                pltpu.make_async_copy(
                    nb, out_hbm.at[c, pl.ds(row - CH, CH)],
                    wsems[(t + 1) % 2]).wait()
            pltpu.async_copy(acc.at[pl.ds(row + CH, CH)], nb,
                             rsems[(t + 1) % 2])
        pltpu.async_copy(b, out_hbm.at[c, pl.ds(row, CH)], wsems[t % 2])
    for t in (nt - 2, nt - 1):
        row = s * RPS + t * CH
        pltpu.make_async_copy(bufs[t % 2], out_hbm.at[c, pl.ds(row, CH)],
                              wsems[t % 2]).wait()


_gs_call = pl.kernel(
    _gs_body,
    out_type=jax.ShapeDtypeStruct((NC, NP, D), jnp.float32),
    mesh=_MESH,
    scratch_types=[
        pltpu.VMEM_SHARED((NP, D), jnp.float32),
        pltpu.VMEM((HP, CH), jnp.int32),
        pltpu.VMEM((HP, CH), jnp.int32),
        pltpu.VMEM((CH, D), jnp.float32),
        pltpu.VMEM((CH, D), jnp.float32),
        pltpu.SemaphoreType.DMA,
        pltpu.SemaphoreType.DMA,
        pltpu.SemaphoreType.DMA,
        pltpu.SemaphoreType.DMA,
    ],
)


def _mm_body(x_ref, w_ref, deg_ref, y_ref):
    dinv = lax.rsqrt(deg_ref[...] + 2.0)
    y_ref[...] = jnp.dot(x_ref[...], w_ref[...],
                         preferred_element_type=jnp.float32) * dinv


def _fin_body(p_ref, y_ref, deg_ref, b_ref, g_ref, be_ref, o_ref):
    dinv = lax.rsqrt(deg_ref[...] + 2.0)
    o = dinv * (p_ref[0] + p_ref[1] + 2.0 * y_ref[...]) + b_ref[...]
    mu = jnp.mean(o, axis=-1, keepdims=True)
    ctr = o - mu
    var = jnp.mean(ctr * ctr, axis=-1, keepdims=True)
    h = ctr * lax.rsqrt(var + 1e-5) * g_ref[...] + be_ref[...]
    o_ref[...] = jnp.maximum(h, 0.0)


_RB = 10000


def kernel(x, edge_index, W, b, gamma, beta):
    pad = EPAD - E
    pad_src = (jnp.arange(pad, dtype=jnp.int32) * 131) % N
    pad_dst = N + jnp.arange(pad, dtype=jnp.int32) % (NP - N)
    ep = jnp.concatenate([edge_index, jnp.stack([pad_src, pad_dst])], axis=1)
    srcp = ep[0].reshape(NW, NCH, CH)
    dstp = ep[1].reshape(NW, NCH, CH)

    degp = _deg_call(dstp)
    deg = (degp[0] + degp[1])[:N].reshape(N, 1)

    y = pl.pallas_call(
        _mm_body,
        grid=(N // _RB,),
        in_specs=[
            pl.BlockSpec((_RB, D), lambda i: (i, 0)),
            pl.BlockSpec((D, D), lambda i: (0, 0)),
            pl.BlockSpec((_RB, 1), lambda i: (i, 0)),
        ],
        out_specs=pl.BlockSpec((_RB, D), lambda i: (i, 0)),
        out_shape=jax.ShapeDtypeStruct((N, D), jnp.float32),
    )(x, W, deg)

    parts = _gs_call(y, srcp, dstp)

    h = pl.pallas_call(
        _fin_body,
        grid=(N // _RB,),
        in_specs=[
            pl.BlockSpec((NC, _RB, D), lambda i: (0, i, 0)),
            pl.BlockSpec((_RB, D), lambda i: (i, 0)),
            pl.BlockSpec((_RB, 1), lambda i: (i, 0)),
            pl.BlockSpec((1, D), lambda i: (0, 0)),
            pl.BlockSpec((1, D), lambda i: (0, 0)),
            pl.BlockSpec((1, D), lambda i: (0, 0)),
        ],
        out_specs=pl.BlockSpec((_RB, D), lambda i: (i, 0)),
        out_shape=jax.ShapeDtypeStruct((N, D), jnp.float32),
    )(parts, y, deg, b.reshape(1, D), gamma.reshape(1, D), beta.reshape(1, D))
    return h


# speedup vs baseline: 1.0323x; 1.0323x over previous
import jax
import jax.numpy as jnp
from jax import lax
from jax.experimental import pallas as pl
from jax.experimental.pallas import tpu as pltpu
from jax.experimental.pallas import tpu_sc as plsc

N = 10000
D = 128
E = 320000
NP = 10240
NC = 2
NS = 16
NW = NC * NS
CH = 128
NCH = 80
HP = 40
EPAD = NW * NCH * CH
RPS = NP // NS

_MESH = plsc.VectorSubcoreMesh(core_axis_name="c", subcore_axis_name="s",
                               num_cores=NC, num_subcores=NS)


def _fill(ref, n, value):
    def body(i, _):
        ref[pl.ds(i * 16, 16)] = jnp.full((16,), value, jnp.float32)
        return 0
    lax.fori_loop(0, n // 16, body, 0)


def _deg_body(dst_hbm, out_hbm, acc, idxv, ones, buf, sd):
    c = lax.axis_index("c")
    s = lax.axis_index("s")
    wid = c * NS + s
    _fill(ones, CH, 1.0)
    _fill(buf, RPS, 0.0)
    pltpu.sync_copy(buf, acc.at[pl.ds(s * RPS, RPS)])
    plsc.subcore_barrier()
    pltpu.sync_copy(dst_hbm.at[wid], idxv)

    def body(j, _):
        pltpu.async_copy(ones, acc.at[idxv.at[j]], sd, add=True)
        return 0
    lax.fori_loop(0, NCH, body, 0)

    def drain(j, _):
        pltpu.make_async_copy(ones, acc.at[idxv.at[j]], sd).wait()
        return 0
    lax.fori_loop(0, NCH, drain, 0)
    plsc.subcore_barrier()
    pltpu.sync_copy(acc.at[pl.ds(s * RPS, RPS)], buf)
    pltpu.sync_copy(buf, out_hbm.at[c, pl.ds(s * RPS, RPS)])


_deg_call = pl.kernel(
    _deg_body,
    out_type=jax.ShapeDtypeStruct((NC, NP), jnp.float32),
    mesh=_MESH,
    scratch_types=[
        pltpu.VMEM_SHARED((NP,), jnp.float32),
        pltpu.VMEM((NCH, CH), jnp.int32),
        pltpu.VMEM((CH,), jnp.float32),
        pltpu.VMEM((RPS,), jnp.float32),
        pltpu.SemaphoreType.DMA,
    ],
)


def _gs_body(y_hbm, src_hbm, dst_hbm, out_hbm, acc, sidx, didx, ra, rb,
             sa, sb, ta, tb):
    c = lax.axis_index("c")
    s = lax.axis_index("s")
    wid = c * NS + s

    pltpu.async_copy(src_hbm.at[wid, pl.ds(0, HP)], sidx, ta)
    pltpu.async_copy(dst_hbm.at[wid, pl.ds(0, HP)], didx, tb)

    def zrow(i, _):
        ra[i // 8, pl.ds((i % 8) * 16, 16)] = jnp.zeros((16,), jnp.float32)
        return 0
    lax.fori_loop(0, CH * 8, zrow, 0)
    for t in range(RPS // CH):
        pltpu.async_copy(ra, acc.at[pl.ds(s * RPS + t * CH, CH)], sa)
    for t in range(RPS // CH):
        pltpu.make_async_copy(ra, acc.at[pl.ds(s * RPS + t * CH, CH)],
                              sa).wait()
    plsc.subcore_barrier()

    def gather(j, buf, sem):
        pltpu.async_copy(y_hbm.at[sidx.at[j, pl.ds(0, 64)]],
                         buf.at[pl.ds(0, 64)], sem)
        pltpu.async_copy(y_hbm.at[sidx.at[j, pl.ds(64, 64)]],
                         buf.at[pl.ds(64, 64)], sem)

    def gwait(j, buf, sem):
        pltpu.make_async_copy(y_hbm.at[sidx.at[j]], buf, sem).wait()

    for p in range(NCH // HP):
        if p == 0:
            pltpu.make_async_copy(src_hbm.at[wid, pl.ds(0, HP)], sidx,
                                  ta).wait()
            pltpu.make_async_copy(dst_hbm.at[wid, pl.ds(0, HP)], didx,
                                  tb).wait()
        else:
            pltpu.sync_copy(src_hbm.at[wid, pl.ds(p * HP, HP)], sidx)
            pltpu.sync_copy(dst_hbm.at[wid, pl.ds(p * HP, HP)], didx)
        gather(0, ra, sa)

        def body(i, _):
            j0 = 2 * i
            j1 = j0 + 1
            gwait(j0, ra, sa)
            gather(j1, rb, sb)
            pltpu.sync_copy(ra, acc.at[didx.at[j0]], add=True)

            @pl.when(j1 + 1 < HP)
            def _():
                gather(j1 + 1, ra, sa)
            gwait(j1, rb, sb)
            pltpu.sync_copy(rb, acc.at[didx.at[j1]], add=True)
            return 0
        lax.fori_loop(0, HP // 2, body, 0)
    plsc.subcore_barrier()
    bufs = [ra, rb]
    rsems = [sa, sb]
    wsems = [ta, tb]
    nt = RPS // CH
    pltpu.async_copy(acc.at[pl.ds(s * RPS, CH)], ra, sa)
    for t in range(nt):
        b = bufs[t % 2]
        nb = bufs[(t + 1) % 2]
        row = s * RPS + t * CH
        pltpu.make_async_copy(acc.at[pl.ds(row, CH)], b, rsems[t % 2]).wait()
        if t + 1 < nt:
            if t >= 1:
                pltpu.make_async_copy(
                    nb, out_hbm.at[c, pl.ds(row - CH, CH)],
                    wsems[(t + 1) % 2]).wait()
            pltpu.async_copy(acc.at[pl.ds(row + CH, CH)], nb,
                             rsems[(t + 1) % 2])
        pltpu.async_copy(b, out_hbm.at[c, pl.ds(row, CH)], wsems[t % 2])
    for t in (nt - 2, nt - 1):
        row = s * RPS + t * CH
        pltpu.make_async_copy(bufs[t % 2], out_hbm.at[c, pl.ds(row, CH)],
                              wsems[t % 2]).wait()


_gs_call = pl.kernel(
    _gs_body,
    out_type=jax.ShapeDtypeStruct((NC, NP, D), jnp.float32),
    mesh=_MESH,
    scratch_types=[
        pltpu.VMEM_SHARED((NP, D), jnp.float32),
        pltpu.VMEM((HP, CH), jnp.int32),
        pltpu.VMEM((HP, CH), jnp.int32),
        pltpu.VMEM((CH, D), jnp.float32),
        pltpu.VMEM((CH, D), jnp.float32),
        pltpu.SemaphoreType.DMA,
        pltpu.SemaphoreType.DMA,
        pltpu.SemaphoreType.DMA,
        pltpu.SemaphoreType.DMA,
    ],
)


def _mm_body(x_ref, w_ref, deg_ref, y_ref):
    dinv = lax.rsqrt(deg_ref[...] + 2.0)
    y_ref[...] = jnp.dot(x_ref[...], w_ref[...],
                         preferred_element_type=jnp.float32) * dinv


def _fin_body(p_ref, y_ref, deg_ref, b_ref, g_ref, be_ref, o_ref):
    dinv = lax.rsqrt(deg_ref[...] + 2.0)
    o = dinv * (p_ref[0] + p_ref[1] + 2.0 * y_ref[...]) + b_ref[...]
    mu = jnp.mean(o, axis=-1, keepdims=True)
    ctr = o - mu
    var = jnp.mean(ctr * ctr, axis=-1, keepdims=True)
    h = ctr * lax.rsqrt(var + 1e-5) * g_ref[...] + be_ref[...]
    o_ref[...] = jnp.maximum(h, 0.0)


_RB = 5000


def kernel(x, edge_index, W, b, gamma, beta):
    pad = EPAD - E
    pad_src = (jnp.arange(pad, dtype=jnp.int32) * 131) % N
    pad_dst = N + jnp.arange(pad, dtype=jnp.int32) % (NP - N)
    ep = jnp.concatenate([edge_index, jnp.stack([pad_src, pad_dst])], axis=1)
    srcp = ep[0].reshape(NW, NCH, CH)
    dstp = ep[1].reshape(NW, NCH, CH)

    degp = _deg_call(dstp)
    deg = (degp[0] + degp[1])[:N].reshape(N, 1)

    y = pl.pallas_call(
        _mm_body,
        grid=(N // _RB,),
        in_specs=[
            pl.BlockSpec((_RB, D), lambda i: (i, 0)),
            pl.BlockSpec((D, D), lambda i: (0, 0)),
            pl.BlockSpec((_RB, 1), lambda i: (i, 0)),
        ],
        out_specs=pl.BlockSpec((_RB, D), lambda i: (i, 0)),
        out_shape=jax.ShapeDtypeStruct((N, D), jnp.float32),
    )(x, W, deg)

    parts = _gs_call(y, srcp, dstp)

    h = pl.pallas_call(
        _fin_body,
        grid=(N // _RB,),
        in_specs=[
            pl.BlockSpec((NC, _RB, D), lambda i: (0, i, 0)),
            pl.BlockSpec((_RB, D), lambda i: (i, 0)),
            pl.BlockSpec((_RB, 1), lambda i: (i, 0)),
            pl.BlockSpec((1, D), lambda i: (0, 0)),
            pl.BlockSpec((1, D), lambda i: (0, 0)),
            pl.BlockSpec((1, D), lambda i: (0, 0)),
        ],
        out_specs=pl.BlockSpec((_RB, D), lambda i: (i, 0)),
        out_shape=jax.ShapeDtypeStruct((N, D), jnp.float32),
    )(parts, y, deg, b.reshape(1, D), gamma.reshape(1, D), beta.reshape(1, D))
    return h


# speedup vs baseline: 1.0330x; 1.0007x over previous
import jax
import jax.numpy as jnp
from jax import lax
from jax.experimental import pallas as pl
from jax.experimental.pallas import tpu as pltpu
from jax.experimental.pallas import tpu_sc as plsc

N = 10000
D = 128
E = 320000
NP = 10240
NC = 2
NS = 16
NW = NC * NS
CH = 128
NCH = 80
HP = 40
EPAD = NW * NCH * CH
RPS = NP // NS

_MESH = plsc.VectorSubcoreMesh(core_axis_name="c", subcore_axis_name="s",
                               num_cores=NC, num_subcores=NS)


def _fill(ref, n, value):
    def body(i, _):
        ref[pl.ds(i * 16, 16)] = jnp.full((16,), value, jnp.float32)
        return 0
    lax.fori_loop(0, n // 16, body, 0)


def _deg_body(dst_hbm, out_hbm, acc, idxv, ones, buf, sd):
    c = lax.axis_index("c")
    s = lax.axis_index("s")
    wid = c * NS + s
    pltpu.async_copy(dst_hbm.at[wid], idxv, sd)
    _fill(ones, CH, 1.0)
    _fill(buf, RPS, 0.0)
    pltpu.sync_copy(buf, acc.at[pl.ds(s * RPS, RPS)])
    plsc.subcore_barrier()
    pltpu.make_async_copy(dst_hbm.at[wid], idxv, sd).wait()

    def body(j, _):
        pltpu.async_copy(ones, acc.at[idxv.at[j]], sd, add=True)
        return 0
    lax.fori_loop(0, NCH, body, 0)

    def drain(j, _):
        pltpu.make_async_copy(ones, acc.at[idxv.at[j]], sd).wait()
        return 0
    lax.fori_loop(0, NCH, drain, 0)
    plsc.subcore_barrier()
    pltpu.sync_copy(acc.at[pl.ds(s * RPS, RPS)], buf)
    pltpu.sync_copy(buf, out_hbm.at[c, pl.ds(s * RPS, RPS)])


_deg_call = pl.kernel(
    _deg_body,
    out_type=jax.ShapeDtypeStruct((NC, NP), jnp.float32),
    mesh=_MESH,
    scratch_types=[
        pltpu.VMEM_SHARED((NP,), jnp.float32),
        pltpu.VMEM((NCH, CH), jnp.int32),
        pltpu.VMEM((CH,), jnp.float32),
        pltpu.VMEM((RPS,), jnp.float32),
        pltpu.SemaphoreType.DMA,
    ],
)


def _gs_body(y_hbm, src_hbm, dst_hbm, out_hbm, acc, sidx, didx, ra, rb,
             sa, sb, ta, tb):
    c = lax.axis_index("c")
    s = lax.axis_index("s")
    wid = c * NS + s

    pltpu.async_copy(src_hbm.at[wid, pl.ds(0, HP)], sidx, ta)
    pltpu.async_copy(dst_hbm.at[wid, pl.ds(0, HP)], didx, tb)

    def zrow(i, _):
        ra[i // 8, pl.ds((i % 8) * 16, 16)] = jnp.zeros((16,), jnp.float32)
        return 0
    lax.fori_loop(0, CH * 8, zrow, 0)
    for t in range(RPS // CH):
        pltpu.async_copy(ra, acc.at[pl.ds(s * RPS + t * CH, CH)], sa)
    for t in range(RPS // CH):
        pltpu.make_async_copy(ra, acc.at[pl.ds(s * RPS + t * CH, CH)],
                              sa).wait()
    plsc.subcore_barrier()

    def gather(j, buf, sem):
        pltpu.async_copy(y_hbm.at[sidx.at[j, pl.ds(0, 64)]],
                         buf.at[pl.ds(0, 64)], sem)
        pltpu.async_copy(y_hbm.at[sidx.at[j, pl.ds(64, 64)]],
                         buf.at[pl.ds(64, 64)], sem)

    def gwait(j, buf, sem):
        pltpu.make_async_copy(y_hbm.at[sidx.at[j]], buf, sem).wait()

    for p in range(NCH // HP):
        if p == 0:
            pltpu.make_async_copy(src_hbm.at[wid, pl.ds(0, HP)], sidx,
                                  ta).wait()
            pltpu.make_async_copy(dst_hbm.at[wid, pl.ds(0, HP)], didx,
                                  tb).wait()
        else:
            pltpu.sync_copy(src_hbm.at[wid, pl.ds(p * HP, HP)], sidx)
            pltpu.sync_copy(dst_hbm.at[wid, pl.ds(p * HP, HP)], didx)
        gather(0, ra, sa)

        def body(i, _):
            j0 = 2 * i
            j1 = j0 + 1
            gwait(j0, ra, sa)
            gather(j1, rb, sb)
            pltpu.sync_copy(ra, acc.at[didx.at[j0]], add=True)

            @pl.when(j1 + 1 < HP)
            def _():
                gather(j1 + 1, ra, sa)
            gwait(j1, rb, sb)
            pltpu.sync_copy(rb, acc.at[didx.at[j1]], add=True)
            return 0
        lax.fori_loop(0, HP // 2, body, 0)
    plsc.subcore_barrier()
    bufs = [ra, rb]
    rsems = [sa, sb]
    wsems = [ta, tb]
    nt = RPS // CH
    pltpu.async_copy(acc.at[pl.ds(s * RPS, CH)], ra, sa)
    for t in range(nt):
        b = bufs[t % 2]
        nb = bufs[(t + 1) % 2]
        row = s * RPS + t * CH
        pltpu.make_async_copy(acc.at[pl.ds(row, CH)], b, rsems[t % 2]).wait()
        if t + 1 < nt:
            if t >= 1:
                pltpu.make_async_copy(
                    nb, out_hbm.at[c, pl.ds(row - CH, CH)],
                    wsems[(t + 1) % 2]).wait()
            pltpu.async_copy(acc.at[pl.ds(row + CH, CH)], nb,
                             rsems[(t + 1) % 2])
        pltpu.async_copy(b, out_hbm.at[c, pl.ds(row, CH)], wsems[t % 2])
    for t in (nt - 2, nt - 1):
        row = s * RPS + t * CH
        pltpu.make_async_copy(bufs[t % 2], out_hbm.at[c, pl.ds(row, CH)],
                              wsems[t % 2]).wait()


_gs_call = pl.kernel(
    _gs_body,
    out_type=jax.ShapeDtypeStruct((NC, NP, D), jnp.float32),
    mesh=_MESH,
    scratch_types=[
        pltpu.VMEM_SHARED((NP, D), jnp.float32),
        pltpu.VMEM((HP, CH), jnp.int32),
        pltpu.VMEM((HP, CH), jnp.int32),
        pltpu.VMEM((CH, D), jnp.float32),
        pltpu.VMEM((CH, D), jnp.float32),
        pltpu.SemaphoreType.DMA,
        pltpu.SemaphoreType.DMA,
        pltpu.SemaphoreType.DMA,
        pltpu.SemaphoreType.DMA,
    ],
)


def _mm_body(x_ref, w_ref, deg_ref, y_ref):
    dinv = lax.rsqrt(deg_ref[...] + 2.0)
    y_ref[...] = jnp.dot(x_ref[...], w_ref[...],
                         preferred_element_type=jnp.float32) * dinv


def _fin_body(p_ref, y_ref, deg_ref, b_ref, g_ref, be_ref, o_ref):
    dinv = lax.rsqrt(deg_ref[...] + 2.0)
    o = dinv * (p_ref[0] + p_ref[1] + 2.0 * y_ref[...]) + b_ref[...]
    mu = jnp.mean(o, axis=-1, keepdims=True)
    ctr = o - mu
    var = jnp.mean(ctr * ctr, axis=-1, keepdims=True)
    h = ctr * lax.rsqrt(var + 1e-5) * g_ref[...] + be_ref[...]
    o_ref[...] = jnp.maximum(h, 0.0)


_RB = 5000


def kernel(x, edge_index, W, b, gamma, beta):
    pad = EPAD - E
    pad_src = (jnp.arange(pad, dtype=jnp.int32) * 131) % N
    pad_dst = N + jnp.arange(pad, dtype=jnp.int32) % (NP - N)
    ep = jnp.concatenate([edge_index, jnp.stack([pad_src, pad_dst])], axis=1)
    srcp = ep[0].reshape(NW, NCH, CH)
    dstp = ep[1].reshape(NW, NCH, CH)

    degp = _deg_call(dstp)
    deg = (degp[0] + degp[1])[:N].reshape(N, 1)

    y = pl.pallas_call(
        _mm_body,
        grid=(N // _RB,),
        in_specs=[
            pl.BlockSpec((_RB, D), lambda i: (i, 0)),
            pl.BlockSpec((D, D), lambda i: (0, 0)),
            pl.BlockSpec((_RB, 1), lambda i: (i, 0)),
        ],
        out_specs=pl.BlockSpec((_RB, D), lambda i: (i, 0)),
        out_shape=jax.ShapeDtypeStruct((N, D), jnp.float32),
    )(x, W, deg)

    parts = _gs_call(y, srcp, dstp)

    h = pl.pallas_call(
        _fin_body,
        grid=(N // _RB,),
        in_specs=[
            pl.BlockSpec((NC, _RB, D), lambda i: (0, i, 0)),
            pl.BlockSpec((_RB, D), lambda i: (i, 0)),
            pl.BlockSpec((_RB, 1), lambda i: (i, 0)),
            pl.BlockSpec((1, D), lambda i: (0, 0)),
            pl.BlockSpec((1, D), lambda i: (0, 0)),
            pl.BlockSpec((1, D), lambda i: (0, 0)),
        ],
        out_specs=pl.BlockSpec((_RB, D), lambda i: (i, 0)),
        out_shape=jax.ShapeDtypeStruct((N, D), jnp.float32),
    )(parts, y, deg, b.reshape(1, D), gamma.reshape(1, D), beta.reshape(1, D))
    return h
